# baseline (speedup 1.0000x reference)
import functools

import jax
import jax.numpy as jnp
from jax import lax
from jax.experimental import pallas as pl
from jax.experimental.pallas import tpu as pltpu
from jax.experimental.pallas import tpu_sc as plsc

NUM_EMB = 8192
DIM = 256
BK = 2048
KB = NUM_EMB // BK
HW = 1024


BN = 128
GN = NUM_EMB // 128


def _dist_argmin_body(x_ref, cb2_ref, x2_ref, w2_ref, idx_ref, loss_ref):
    r = pl.program_id(0)

    xt = x_ref[0]
    cross2 = lax.dot_general(xt, cb2_ref[...], (((0,), (1,)), ((), ())),
                             preferred_element_type=jnp.float32)
    x2 = x2_ref[0]
    w2 = w2_ref[...]

    runmin = None
    runidx = None
    big = jnp.int32(2**31 - 1)
    for g in range(GN):
        sl = slice(g * 128, (g + 1) * 128)
        d2 = (x2 + w2[:, sl]) - cross2[:, sl]
        dist = jnp.sqrt(jnp.maximum(d2, 0.0))
        ig = lax.broadcasted_iota(jnp.int32, (1, 128), 1) + g * 128
        if g == 0:
            runmin = dist
            runidx = jnp.broadcast_to(ig, (BN, 128))
        else:
            upd = dist < runmin
            runmin = jnp.where(upd, dist, runmin)
            runidx = jnp.where(upd, ig, runidx)

    lmin = jnp.min(runmin, axis=1, keepdims=True)
    lidx = jnp.min(jnp.where(runmin == lmin, runidx, big),
                   axis=1, keepdims=True)
    idx_ref[0] = lidx

    s = jnp.sum(lmin * lmin, keepdims=True)

    @pl.when(r == 0)
    def _():
        loss_ref[...] = s

    @pl.when(r > 0)
    def _():
        loss_ref[...] = loss_ref[...] + s


def _dist_argmin(xr, cb2, x2, w2):
    B = xr.shape[0]
    grid = (B * HW // BN,)
    nsub = HW // BN
    out = pl.pallas_call(
        _dist_argmin_body,
        grid=grid,
        in_specs=[
            pl.BlockSpec((1, DIM, BN), lambda r: (r // nsub, 0, r % nsub)),
            pl.BlockSpec((NUM_EMB, DIM), lambda r: (0, 0)),
            pl.BlockSpec((1, BN, 1), lambda r: (r // nsub, r % nsub, 0)),
            pl.BlockSpec((1, NUM_EMB), lambda r: (0, 0)),
        ],
        out_specs=[
            pl.BlockSpec((1, BN, 1), lambda r: (r // nsub, r % nsub, 0)),
            pl.BlockSpec((1, 1), lambda r: (0, 0)),
        ],
        out_shape=[
            jax.ShapeDtypeStruct((B, HW, 1), jnp.int32),
            jax.ShapeDtypeStruct((1, 1), jnp.float32),
        ],
    )(xr, cb2, x2, w2)
    return out


_SC_WORKERS = 32
_BPW = (8 * HW) // _SC_WORKERS


@functools.lru_cache(maxsize=1)
def _make_sc_gather():
    @functools.partial(
        pl.kernel,
        mesh=plsc.VectorSubcoreMesh(core_axis_name="c", subcore_axis_name="s"),
        out_type=jax.ShapeDtypeStruct((8 * HW, DIM), jnp.float32),
        scratch_types=[
            pltpu.VMEM((_BPW,), jnp.int32),
            pltpu.VMEM((_BPW, DIM), jnp.float32),
            pltpu.SemaphoreType.DMA,
        ],
    )
    def _sc_gather(table_hbm, idx_hbm, out_hbm, idx_v, rows_v, sem):
        wid = lax.axis_index("s") * 2 + lax.axis_index("c")
        base = wid * _BPW
        pltpu.sync_copy(idx_hbm.at[pl.ds(base, _BPW)], idx_v)
        pltpu.async_copy(table_hbm.at[idx_v], rows_v, sem).wait()
        pltpu.sync_copy(rows_v, out_hbm.at[pl.ds(base, _BPW)])

    return _sc_gather


def kernel(x, codebook):
    B, C, H, W = x.shape
    hw = H * W
    xr = x.reshape(B, C, hw)
    xf = jnp.transpose(xr, (0, 2, 1))
    x2 = jnp.sum(xf ** 2, axis=-1, keepdims=True)
    w2 = jnp.sum(codebook ** 2, axis=-1).reshape(1, NUM_EMB)
    cb2 = codebook * 2.0

    idx, loss_sum = _dist_argmin(xr, cb2, x2, w2)

    idx_flat = idx.reshape(B * hw)
    quant = _make_sc_gather()(codebook, idx_flat)

    quant_out = jnp.transpose(quant.reshape(B, hw, C), (0, 2, 1)).reshape(
        B, C, H, W)
    loss = loss_sum[0, 0] / jnp.float32(B * hw * C)
    indices = idx.reshape(B, H, W)
    return quant_out, loss, loss, indices
